# Initial kernel scaffold; baseline (speedup 1.0000x reference)
import functools

import jax
import jax.numpy as jnp
from jax import lax
from jax.experimental import pallas as pl
from jax.experimental.pallas import tpu as pltpu
from jax.experimental.pallas import tpu_sc as plsc

N_NODES = 10000
D = 128

NC = 2
NS = 16
NW = NC * NS
L = 16

CHUNK = 128
ROWS_PER_TILE = N_NODES // NS
ZROWS = 125


def _mm_kernel(x_ref, w_ref, o_ref):
    o_ref[...] = jnp.dot(x_ref[...], w_ref[...],
                         preferred_element_type=jnp.float32)


def _mid_kernel(p0_ref, p1_ref, w_ref, o_ref):
    h = jnp.maximum(p0_ref[...] + p1_ref[...], 0.0)
    o_ref[...] = jnp.dot(h, w_ref[...], preferred_element_type=jnp.float32)


def _relu_sum_kernel(p0_ref, p1_ref, o_ref):
    o_ref[...] = jnp.maximum(p0_ref[...] + p1_ref[...], 0.0)


_NBLK = 20
_BLK = N_NODES // _NBLK


def _tc_matmul(x, w):
    return pl.pallas_call(
        _mm_kernel,
        grid=(_NBLK,),
        in_specs=[pl.BlockSpec((_BLK, D), lambda i: (i, 0)),
                  pl.BlockSpec((D, D), lambda i: (0, 0))],
        out_specs=pl.BlockSpec((_BLK, D), lambda i: (i, 0)),
        out_shape=jax.ShapeDtypeStruct((N_NODES, D), jnp.float32),
    )(x, w)


def _tc_mid(p0, p1, w):
    return pl.pallas_call(
        _mid_kernel,
        grid=(_NBLK,),
        in_specs=[pl.BlockSpec((_BLK, D), lambda i: (i, 0)),
                  pl.BlockSpec((_BLK, D), lambda i: (i, 0)),
                  pl.BlockSpec((D, D), lambda i: (0, 0))],
        out_specs=pl.BlockSpec((_BLK, D), lambda i: (i, 0)),
        out_shape=jax.ShapeDtypeStruct((N_NODES, D), jnp.float32),
    )(p0, p1, w)


def _tc_relu_sum(p0, p1):
    return pl.pallas_call(
        _relu_sum_kernel,
        grid=(_NBLK,),
        in_specs=[pl.BlockSpec((_BLK, D), lambda i: (i, 0)),
                  pl.BlockSpec((_BLK, D), lambda i: (i, 0))],
        out_specs=pl.BlockSpec((_BLK, D), lambda i: (i, 0)),
        out_shape=jax.ShapeDtypeStruct((N_NODES, D), jnp.float32),
    )(p0, p1)


def _make_sc_spmm(cpt):
    mesh = plsc.VectorSubcoreMesh(core_axis_name="c", subcore_axis_name="s")

    @functools.partial(
        pl.kernel,
        out_type=jax.ShapeDtypeStruct((NC, N_NODES, D), jnp.float32),
        mesh=mesh,
        scratch_types=[
            pltpu.VMEM((cpt, CHUNK), jnp.int32),
            pltpu.VMEM((cpt, CHUNK), jnp.int32),
            pltpu.VMEM((cpt, CHUNK), jnp.float32),
            pltpu.VMEM((CHUNK, D), jnp.float32),
            pltpu.VMEM((ZROWS, D), jnp.float32),
            pltpu.VMEM_SHARED((N_NODES, D), jnp.float32),
            pltpu.SemaphoreType.DMA,
        ],
    )
    def spmm(h_hbm, src_hbm, dst_hbm, w_hbm, out_hbm,
             src_v, dst_v, w_v, rows_v, zero_v, acc_sh, sem):
        cid = lax.axis_index("c")
        sid = lax.axis_index("s")
        wid = cid * NS + sid

        pltpu.sync_copy(src_hbm.at[wid], src_v)
        pltpu.sync_copy(dst_hbm.at[wid], dst_v)
        pltpu.sync_copy(w_hbm.at[wid], w_v)

        zvec = jnp.zeros((L,), jnp.float32)

        def zero_body(r, carry):
            for j in range(D // L):
                zero_v[r, pl.ds(j * L, L)] = zvec
            return carry
        lax.fori_loop(0, ZROWS, zero_body, 0)
        row0 = sid * ROWS_PER_TILE
        for z in range(ROWS_PER_TILE // ZROWS):
            pltpu.sync_copy(zero_v, acc_sh.at[pl.ds(row0 + z * ZROWS, ZROWS)])
        plsc.subcore_barrier()

        def chunk_body(c, carry):
            pltpu.async_copy(h_hbm.at[src_v.at[c]], rows_v, sem).wait()

            def edge_body(e, carry2):
                wvec = plsc.load_gather(
                    w_v, [jnp.full((L,), c, jnp.int32),
                          jnp.full((L,), e, jnp.int32)])
                for j in range(D // L):
                    sl = pl.ds(j * L, L)
                    rows_v[e, sl] = rows_v[e, sl] * wvec
                return carry2
            lax.fori_loop(0, CHUNK, edge_body, 0)

            pltpu.sync_copy(rows_v, acc_sh.at[dst_v.at[c]], add=True)
            return carry
        lax.fori_loop(0, cpt, chunk_body, 0)
        plsc.subcore_barrier()

        pltpu.sync_copy(acc_sh.at[pl.ds(row0, ROWS_PER_TILE)],
                        out_hbm.at[cid, pl.ds(row0, ROWS_PER_TILE)])

    return spmm


def kernel(x, edge_index, edge_weight, nodes_mask, W0, W1):
    del nodes_mask
    n_edges = edge_index.shape[1]
    per_tile = -(-n_edges // (NW * CHUNK)) * CHUNK
    cpt = per_tile // CHUNK
    ep = NW * per_tile
    pad = ep - n_edges

    src = edge_index[0].astype(jnp.int32)
    dst = edge_index[1].astype(jnp.int32)
    w = edge_weight.astype(jnp.float32)
    if pad:
        zpad = jnp.zeros((pad,), jnp.int32)
        src = jnp.concatenate([src, zpad])
        dst = jnp.concatenate([dst, zpad])
        w = jnp.concatenate([w, jnp.zeros((pad,), jnp.float32)])
    src = src.reshape(NW, cpt, CHUNK)
    dst = dst.reshape(NW, cpt, CHUNK)
    w = w.reshape(NW, cpt, CHUNK)

    spmm = _make_sc_spmm(cpt)

    h0 = _tc_matmul(x, W0)
    p = spmm(h0, src, dst, w)
    h1 = _tc_mid(p[0], p[1], W1)
    p2 = spmm(h1, src, dst, w)
    return _tc_relu_sum(p2[0], p2[1])


# speedup vs baseline: 4.3304x; 4.3304x over previous
import functools

import numpy as np
import jax
import jax.numpy as jnp
from jax import lax
from jax.experimental import pallas as pl
from jax.experimental.pallas import tpu as pltpu
from jax.experimental.pallas import tpu_sc as plsc

N_NODES = 10000
N_PAD = 10240
D = 128

NC = 2
NS = 16
NW = NC * NS
L = 16

CHUNK = 128

_GATHER_DNUMS = jax.lax.GatherDimensionNumbers(
    offset_dims=(), collapsed_slice_dims=(0,), start_index_map=(0,))
ROWS_PER_TILE = N_PAD // NS
ZROWS = 128


def _mm_kernel(x_ref, w_ref, o_ref):
    o_ref[...] = jnp.dot(x_ref[...], w_ref[...],
                         preferred_element_type=jnp.float32)


def _mid_kernel(p_ref, w_ref, o_ref):
    h = jnp.maximum(p_ref[0] + p_ref[1], 0.0)
    o_ref[...] = jnp.dot(h, w_ref[...], preferred_element_type=jnp.float32)


def _relu_sum_kernel(p_ref, o_ref):
    o_ref[...] = jnp.maximum(p_ref[0] + p_ref[1], 0.0)


_NBLK = 25
_BLK = N_NODES // _NBLK


def _tc_matmul(x, w):
    return pl.pallas_call(
        _mm_kernel,
        grid=(_NBLK,),
        in_specs=[pl.BlockSpec((_BLK, D), lambda i: (i, 0)),
                  pl.BlockSpec((D, D), lambda i: (0, 0))],
        out_specs=pl.BlockSpec((_BLK, D), lambda i: (i, 0)),
        out_shape=jax.ShapeDtypeStruct((N_NODES, D), jnp.float32),
    )(x, w)


def _tc_mid(p, w):
    return pl.pallas_call(
        _mid_kernel,
        grid=(_NBLK,),
        in_specs=[pl.BlockSpec((NC, _BLK, D), lambda i: (0, i, 0)),
                  pl.BlockSpec((D, D), lambda i: (0, 0))],
        out_specs=pl.BlockSpec((_BLK, D), lambda i: (i, 0)),
        out_shape=jax.ShapeDtypeStruct((N_NODES, D), jnp.float32),
    )(p, w)


def _tc_relu_sum(p):
    return pl.pallas_call(
        _relu_sum_kernel,
        grid=(_NBLK,),
        in_specs=[pl.BlockSpec((NC, _BLK, D), lambda i: (0, i, 0))],
        out_specs=pl.BlockSpec((_BLK, D), lambda i: (i, 0)),
        out_shape=jax.ShapeDtypeStruct((N_NODES, D), jnp.float32),
    )(p)


def _make_sc_spmm(cpt):
    mesh = plsc.VectorSubcoreMesh(core_axis_name="c", subcore_axis_name="s")

    @functools.partial(
        pl.kernel,
        out_type=jax.ShapeDtypeStruct((NC, N_PAD, D), jnp.float32),
        mesh=mesh,
        scratch_types=[
            pltpu.VMEM((cpt, CHUNK), jnp.int32),
            pltpu.VMEM((cpt, CHUNK), jnp.int32),
            pltpu.VMEM((cpt * CHUNK,), jnp.float32),
            pltpu.VMEM((CHUNK, D), jnp.float32),
            pltpu.VMEM_SHARED((N_PAD, D), jnp.float32),
            pltpu.SemaphoreType.DMA,
        ],
    )
    def spmm(h_hbm, src_hbm, dst_hbm, w_hbm, out_hbm,
             src_v, dst_v, w_v, rows_v, acc_sh, sem):
        cid = lax.axis_index("c")
        sid = lax.axis_index("s")
        wid = cid * NS + sid

        pltpu.sync_copy(src_hbm.at[wid], src_v)
        pltpu.sync_copy(dst_hbm.at[wid], dst_v)
        pltpu.sync_copy(w_hbm.at[wid], w_v)

        zvec = jnp.zeros((L,), jnp.float32)

        def zero_body(r, carry):
            for j in range(D // L):
                rows_v[r, pl.ds(j * L, L)] = zvec
            return carry
        lax.fori_loop(0, ZROWS, zero_body, 0)
        row0 = sid * ROWS_PER_TILE
        for z in range(ROWS_PER_TILE // ZROWS):
            pltpu.sync_copy(rows_v, acc_sh.at[pl.ds(row0 + z * ZROWS, ZROWS)])
        plsc.subcore_barrier()

        zlanes = lax.iota(jnp.int32, L) * 0

        def chunk_body(c, carry):
            pltpu.async_copy(h_hbm.at[src_v.at[c]], rows_v, sem).wait()

            def grp_body(g, carry2):
                wgrp = w_v[pl.ds(c * CHUNK + g * L, L)]
                for e16 in range(L):
                    idx = (zlanes + e16).reshape(L, 1)
                    wvec = lax.gather(
                        wgrp, idx,
                        _GATHER_DNUMS, slice_sizes=(1,),
                        mode=lax.GatherScatterMode.PROMISE_IN_BOUNDS)
                    e = g * L + e16
                    for j in range(D // L):
                        sl = pl.ds(j * L, L)
                        rows_v[e, sl] = rows_v[e, sl] * wvec
                return carry2
            lax.fori_loop(0, CHUNK // L, grp_body, 0)

            pltpu.sync_copy(rows_v, acc_sh.at[dst_v.at[c]], add=True)
            return carry
        lax.fori_loop(0, cpt, chunk_body, 0)
        plsc.subcore_barrier()

        pltpu.sync_copy(acc_sh.at[pl.ds(row0, ROWS_PER_TILE)],
                        out_hbm.at[cid, pl.ds(row0, ROWS_PER_TILE)])

    return spmm


def kernel(x, edge_index, edge_weight, nodes_mask, W0, W1):
    del nodes_mask
    n_edges = edge_index.shape[1]
    per_tile = -(-n_edges // (NW * CHUNK)) * CHUNK
    cpt = per_tile // CHUNK
    ep = NW * per_tile
    pad = ep - n_edges

    src = edge_index[0].astype(jnp.int32)
    dst = edge_index[1].astype(jnp.int32)
    w = edge_weight.astype(jnp.float32)
    if pad:
        zpad = jnp.zeros((pad,), jnp.int32)
        src = jnp.concatenate([src, zpad])
        dst = jnp.concatenate([dst, zpad])
        w = jnp.concatenate([w, jnp.zeros((pad,), jnp.float32)])
    src = src.reshape(NW, cpt, CHUNK)
    dst = dst.reshape(NW, cpt, CHUNK)
    w = w.reshape(NW, cpt * CHUNK)

    spmm = _make_sc_spmm(cpt)

    h0 = _tc_matmul(x, W0)
    p = spmm(h0, src, dst, w)
    h1 = _tc_mid(p, W1)
    p2 = spmm(h1, src, dst, w)
    return _tc_relu_sum(p2)
